# baseline (speedup 1.0000x reference)
import functools

import jax
import jax.numpy as jnp
from jax.experimental import pallas as pl

N = 8192
C = 256
NS = 16
S = 8
CS = C // S
M_STAT = N * NS

HIGH = jax.lax.Precision.HIGHEST


def _proj_kernel(x_ref, wq_ref, bq_ref, wk_ref, bk_ref, wv_ref, bv_ref,
                 xq_ref, xk_ref, xv_ref):
    x = x_ref[...]
    xq_ref[...] = jnp.dot(x, wq_ref[...].T, precision=HIGH) + bq_ref[...]
    xk_ref[...] = jnp.dot(x, wk_ref[...].T, precision=HIGH) + bk_ref[...]
    xv_ref[...] = jnp.dot(x, wv_ref[...].T, precision=HIGH) + bv_ref[...]


def _prstats_kernel(gx_ref, gy_ref, gz_ref, wp1_ref, bp1_ref,
                    p0_ref, p1_ref, p2_ref, acc_ref):
    g = (gx_ref[...], gy_ref[...], gz_ref[...])
    outs = (p0_ref, p1_ref, p2_ref)

    @pl.when(pl.program_id(0) == 0)
    def _():
        acc_ref[...] = jnp.zeros_like(acc_ref)

    for c in range(3):
        praw = bp1_ref[c:c + 1, 0:1]
        for d in range(3):
            praw = praw + g[d] * wp1_ref[c:c + 1, d:d + 1]
        outs[c][...] = praw
        acc_ref[2 * c:2 * c + 1, :] += praw.sum(axis=0, keepdims=True)
        acc_ref[2 * c + 1:2 * c + 2, :] += (praw * praw).sum(axis=0, keepdims=True)


def _pr_block(praw_refs, ab1_ref, wp2t_ref, bp2_ref, bn):
    pr = jnp.broadcast_to(bp2_ref[...][None, None, :], (bn, NS, C))
    for d in range(3):
        prh_d = jax.nn.relu(praw_refs[d][...] * ab1_ref[0:1, d:d + 1]
                            + ab1_ref[1:2, d:d + 1])
        pr = pr + prh_d[:, :, None] * wp2t_ref[d:d + 1, :][None, :, :]
    return pr.reshape(bn * NS, C)


def _wstats_kernel(p0_ref, p1_ref, p2_ref, gk_ref, xq_ref, wp2t_ref, bp2_ref,
                   ab1_ref, acc_ref):
    bn = gk_ref.shape[0]
    pr = _pr_block((p0_ref, p1_ref, p2_ref), ab1_ref, wp2t_ref, bp2_ref, bn)
    gk = gk_ref[...].reshape(bn * NS, C)
    xq = xq_ref[...]
    w_raw = gk - jnp.broadcast_to(xq[:, None, :], (bn, NS, C)).reshape(bn * NS, C) + pr

    @pl.when(pl.program_id(0) == 0)
    def _():
        acc_ref[...] = jnp.zeros_like(acc_ref)

    acc_ref[...] += jnp.stack([w_raw.sum(axis=0), (w_raw * w_raw).sum(axis=0)])


def _w1_kernel(p0_ref, p1_ref, p2_ref, gk_ref, xq_ref, wp2t_ref, bp2_ref,
               ab1_ref, ab2_ref, ww1_ref, bw1_ref, w1_ref, acc_ref):
    bn = gk_ref.shape[0]
    pr = _pr_block((p0_ref, p1_ref, p2_ref), ab1_ref, wp2t_ref, bp2_ref, bn)
    gk = gk_ref[...].reshape(bn * NS, C)
    xq = xq_ref[...]
    w_raw = gk - jnp.broadcast_to(xq[:, None, :], (bn, NS, C)).reshape(bn * NS, C) + pr
    wh = jax.nn.relu(w_raw * ab2_ref[0:1, :] + ab2_ref[1:2, :])
    w1 = jnp.dot(wh, ww1_ref[...].T, precision=HIGH) + bw1_ref[...]
    w1_ref[...] = w1.reshape(bn, NS, CS)

    @pl.when(pl.program_id(0) == 0)
    def _():
        acc_ref[...] = jnp.zeros_like(acc_ref)

    acc_ref[...] += jnp.stack([w1.sum(axis=0), (w1 * w1).sum(axis=0)])


def _outsum_kernel(w1_ref, gv_ref, p0_ref, p1_ref, p2_ref, wp2t_ref, bp2_ref,
                   ab1_ref, ab3_ref, ww2_ref, bw2_ref, o_ref):
    bn = w1_ref.shape[0]
    w1 = w1_ref[...].reshape(bn * NS, CS)
    wh = jax.nn.relu(w1 * ab3_ref[0:1, :] + ab3_ref[1:2, :])
    w2 = jnp.dot(wh, ww2_ref[...].T, precision=HIGH) + bw2_ref[...]
    w2 = w2.reshape(bn, NS, CS)
    w2 = w2 - jnp.max(w2, axis=1, keepdims=True)
    e = jnp.exp(w2)
    w = e / jnp.sum(e, axis=1, keepdims=True)

    pr = _pr_block((p0_ref, p1_ref, p2_ref), ab1_ref, wp2t_ref, bp2_ref, bn)
    t = (gv_ref[...].reshape(bn * NS, C) + pr).reshape(bn, NS, S, CS) * w[:, :, None, :]
    o_ref[...] = t.sum(axis=1).reshape(bn, C)


def _affine(acc, gamma, beta, eps=1e-5):
    mean = acc[0] / M_STAT
    var = acc[1] / M_STAT - mean * mean
    a = gamma / jnp.sqrt(var + eps)
    return jnp.stack([a, beta - mean * a])


def kernel(p, x, o, edges, boundary, Wq, bq, Wk, bk, Wv, bv, Wp1, bp1, gp1,
           betap1, Wp2, bp2, gw1, betaw1, Ww1, bw1l, gw2, betaw2, Ww2, bw2l):
    f32 = jnp.float32

    BP = 1024
    xq, xk, xv = pl.pallas_call(
        _proj_kernel,
        grid=(N // BP,),
        in_specs=[
            pl.BlockSpec((BP, C), lambda i: (i, 0)),
            pl.BlockSpec((C, C), lambda i: (0, 0)),
            pl.BlockSpec((C,), lambda i: (0,)),
            pl.BlockSpec((C, C), lambda i: (0, 0)),
            pl.BlockSpec((C,), lambda i: (0,)),
            pl.BlockSpec((C, C), lambda i: (0, 0)),
            pl.BlockSpec((C,), lambda i: (0,)),
        ],
        out_specs=[pl.BlockSpec((BP, C), lambda i: (i, 0))] * 3,
        out_shape=[jax.ShapeDtypeStruct((N, C), f32)] * 3,
    )(x, Wq, bq, Wk, bk, Wv, bv)

    sq = (p * p).sum(-1)
    d2 = sq[:, None] + sq[None, :] - 2.0 * (p @ p.T)
    _, idx = jax.lax.top_k(-d2, NS)

    gp = p[idx]
    gx = gp[:, :, 0] - p[:, 0:1]
    gy = gp[:, :, 1] - p[:, 1:2]
    gz = gp[:, :, 2] - p[:, 2:3]
    gk = xk[idx]
    gv = xv[idx]
    Wp1f = Wp1.astype(f32)
    bp1c = jnp.broadcast_to(bp1[:, None], (3, 3)).astype(f32)
    Wp2T = Wp2.T

    BA = 2048
    nsspec = lambda: pl.BlockSpec((BA, NS), lambda i: (i, 0))
    p0, p1, p2, accA = pl.pallas_call(
        _prstats_kernel,
        grid=(N // BA,),
        in_specs=[
            nsspec(), nsspec(), nsspec(),
            pl.BlockSpec((3, 3), lambda i: (0, 0)),
            pl.BlockSpec((3, 3), lambda i: (0, 0)),
        ],
        out_specs=[
            nsspec(), nsspec(), nsspec(),
            pl.BlockSpec((6, NS), lambda i: (0, 0)),
        ],
        out_shape=[jax.ShapeDtypeStruct((N, NS), f32)] * 3
        + [jax.ShapeDtypeStruct((6, NS), f32)],
    )(gx, gy, gz, Wp1f, bp1c)
    acc1 = jnp.stack([accA[0::2].sum(axis=1), accA[1::2].sum(axis=1)], axis=0)
    ab1 = _affine(acc1, gp1, betap1)

    BB = 256
    pspec = lambda: pl.BlockSpec((BB, NS), lambda i: (i, 0))
    acc2 = pl.pallas_call(
        _wstats_kernel,
        grid=(N // BB,),
        in_specs=[
            pspec(), pspec(), pspec(),
            pl.BlockSpec((BB, NS, C), lambda i: (i, 0, 0)),
            pl.BlockSpec((BB, C), lambda i: (i, 0)),
            pl.BlockSpec((3, C), lambda i: (0, 0)),
            pl.BlockSpec((C,), lambda i: (0,)),
            pl.BlockSpec((2, 3), lambda i: (0, 0)),
        ],
        out_specs=pl.BlockSpec((2, C), lambda i: (0, 0)),
        out_shape=jax.ShapeDtypeStruct((2, C), f32),
    )(p0, p1, p2, gk, xq, Wp2T, bp2, ab1)
    ab2 = _affine(acc2, gw1, betaw1)

    w1, acc3 = pl.pallas_call(
        _w1_kernel,
        grid=(N // BB,),
        in_specs=[
            pspec(), pspec(), pspec(),
            pl.BlockSpec((BB, NS, C), lambda i: (i, 0, 0)),
            pl.BlockSpec((BB, C), lambda i: (i, 0)),
            pl.BlockSpec((3, C), lambda i: (0, 0)),
            pl.BlockSpec((C,), lambda i: (0,)),
            pl.BlockSpec((2, 3), lambda i: (0, 0)),
            pl.BlockSpec((2, C), lambda i: (0, 0)),
            pl.BlockSpec((CS, C), lambda i: (0, 0)),
            pl.BlockSpec((CS,), lambda i: (0,)),
        ],
        out_specs=[
            pl.BlockSpec((BB, NS, CS), lambda i: (i, 0, 0)),
            pl.BlockSpec((2, CS), lambda i: (0, 0)),
        ],
        out_shape=[
            jax.ShapeDtypeStruct((N, NS, CS), f32),
            jax.ShapeDtypeStruct((2, CS), f32),
        ],
    )(p0, p1, p2, gk, xq, Wp2T, bp2, ab1, ab2, Ww1, bw1l)
    ab3 = _affine(acc3, gw2, betaw2)

    out = pl.pallas_call(
        _outsum_kernel,
        grid=(N // BB,),
        in_specs=[
            pl.BlockSpec((BB, NS, CS), lambda i: (i, 0, 0)),
            pl.BlockSpec((BB, NS, C), lambda i: (i, 0, 0)),
            pspec(), pspec(), pspec(),
            pl.BlockSpec((3, C), lambda i: (0, 0)),
            pl.BlockSpec((C,), lambda i: (0,)),
            pl.BlockSpec((2, 3), lambda i: (0, 0)),
            pl.BlockSpec((2, CS), lambda i: (0, 0)),
            pl.BlockSpec((CS, CS), lambda i: (0, 0)),
            pl.BlockSpec((CS,), lambda i: (0,)),
        ],
        out_specs=pl.BlockSpec((BB, C), lambda i: (i, 0)),
        out_shape=jax.ShapeDtypeStruct((N, C), f32),
    )(w1, gv, p0, p1, p2, Wp2T, bp2, ab1, ab3, Ww2, bw2l)
    return out


# speedup vs baseline: 2.8262x; 2.8262x over previous
import functools

import jax
import jax.numpy as jnp
from jax import lax
from jax.experimental import pallas as pl
from jax.experimental.pallas import tpu as pltpu
from jax.experimental.pallas import tpu_sc as plsc

N = 8192
C = 256
NS = 16
S = 8
CS = C // S
M_STAT = N * NS

HIGH = jax.lax.Precision.HIGHEST


NG = 128
G = 64


def _knnprep_kernel(pp_ref, ppt_ref, sqc_ref, sqr_ref, d2_ref, gm_ref):
    br = pp_ref.shape[0]
    mm = jnp.dot(pp_ref[...].astype(jnp.bfloat16),
                 ppt_ref[...].astype(jnp.bfloat16),
                 preferred_element_type=jnp.float32)
    d2 = (sqc_ref[...][:, None] + sqr_ref[...]) - 2.0 * mm
    d2_ref[...] = d2
    gm_ref[...] = jnp.min(d2.reshape(br, NG, G), axis=2)


_NC = 2
_NSUB = 16
_NW = _NC * _NSUB
_RW = N // _NW


def _merge16(av, ai, bv, bi, resort=True):
    rbv = lax.rev(bv, (0,))
    rbi = lax.rev(bi, (0,))
    m = av <= rbv
    lov = jnp.where(m, av, rbv)
    loi = jnp.where(m, ai, rbi)
    if resort:
        sv, si = plsc.sort_key_val(lov, loi)
        return sv, si
    return lov, loi


def _sc_select_kernel(gm_hbm, d2s_hbm, idx_hbm, gm_v, slab_v, grp_v, gidx_v,
                      idx_v, sem):
    nc = lax.axis_index("c")
    ns = lax.axis_index("s")
    wid = ns * _NC + nc
    base = wid * _RW
    iota = lax.broadcasted_iota(jnp.int32, (16,), 0)

    pltpu.sync_copy(gm_hbm.at[pl.ds(base * NG, _RW * NG)], gm_v)

    def row_body(r, _):
        pairs = []
        for c in range(0, 8, 2):
            av, ai = plsc.sort_key_val(gm_v[pl.ds(r * NG + c * 16, 16)],
                                       c * 16 + iota)
            bv, bi = plsc.sort_key_val(gm_v[pl.ds(r * NG + (c + 1) * 16, 16)],
                                       (c + 1) * 16 + iota)
            pairs.append(_merge16(av, ai, bv, bi))
        m01 = _merge16(*pairs[0], *pairs[1])
        m23 = _merge16(*pairs[2], *pairs[3])
        gv_, gi = _merge16(*m01, *m23)
        grp_v[...] = gi
        gidx_v[...] = (base + r) * NG + gi
        pltpu.async_copy(d2s_hbm.at[gidx_v], slab_v, sem).wait()
        cv, ci = plsc.sort_key_val(slab_v[0, pl.ds(0, 16)], iota)
        for t in range(16):
            for c in range(4):
                if t == 0 and c == 0:
                    continue
                pos = t * 64 + c * 16
                ch = slab_v[t, pl.ds(c * 16, 16)]
                mx = jnp.max(cv)

                def _do(cv=cv, ci=ci, ch=ch, pos=pos):
                    sv, si = plsc.sort_key_val(ch, pos + iota)
                    mv, mi = _merge16(cv, ci, sv, si)
                    return mv, mi

                def _skip(cv=cv, ci=ci):
                    return cv, ci

                cv, ci = lax.cond(jnp.min(ch) < mx, _do, _skip)
        t_of = ci >> 6
        l_of = ci & 63
        gsel = plsc.load_gather(grp_v, [t_of])
        idx_v[pl.ds(r * NS, NS)] = gsel * G + l_of
        return ()

    lax.fori_loop(0, _RW, row_body, ())
    pltpu.sync_copy(idx_v, idx_hbm.at[pl.ds(base * NS, _RW * NS)])


def _sc_select(gm, d2):
    d2s = d2.reshape(N * NG, G)
    gmf = gm.reshape(N * NG)
    mesh = plsc.VectorSubcoreMesh(core_axis_name="c", subcore_axis_name="s")
    fn = pl.kernel(
        _sc_select_kernel,
        out_type=jax.ShapeDtypeStruct((N * NS,), jnp.int32),
        mesh=mesh,
        scratch_types=[
            pltpu.VMEM((_RW * NG,), jnp.float32),
            pltpu.VMEM((16, G), jnp.float32),
            pltpu.VMEM((16,), jnp.int32),
            pltpu.VMEM((16,), jnp.int32),
            pltpu.VMEM((_RW * NS,), jnp.int32),
            pltpu.SemaphoreType.DMA,
        ],
        compiler_params=pltpu.CompilerParams(needs_layout_passes=False,
                                             use_tc_tiling_on_sc=False),
    )
    return fn(gmf, d2s).reshape(N, NS)


def _proj_kernel(x_ref, wq_ref, bq_ref, wk_ref, bk_ref, wv_ref, bv_ref,
                 xq_ref, xk_ref, xv_ref):
    x = x_ref[...]
    xq_ref[...] = jnp.dot(x, wq_ref[...].T, precision=HIGH) + bq_ref[...]
    xk_ref[...] = jnp.dot(x, wk_ref[...].T, precision=HIGH) + bk_ref[...]
    xv_ref[...] = jnp.dot(x, wv_ref[...].T, precision=HIGH) + bv_ref[...]


def _prstats_kernel(gx_ref, gy_ref, gz_ref, wp1_ref, bp1_ref,
                    p0_ref, p1_ref, p2_ref, acc_ref):
    g = (gx_ref[...], gy_ref[...], gz_ref[...])
    outs = (p0_ref, p1_ref, p2_ref)

    @pl.when(pl.program_id(0) == 0)
    def _():
        acc_ref[...] = jnp.zeros_like(acc_ref)

    for c in range(3):
        praw = bp1_ref[c:c + 1, 0:1]
        for d in range(3):
            praw = praw + g[d] * wp1_ref[c:c + 1, d:d + 1]
        outs[c][...] = praw
        acc_ref[2 * c:2 * c + 1, :] += praw.sum(axis=0, keepdims=True)
        acc_ref[2 * c + 1:2 * c + 2, :] += (praw * praw).sum(axis=0, keepdims=True)


def _pr_block(praw_refs, ab1_ref, wp2t_ref, bp2_ref, bn):
    pr = jnp.broadcast_to(bp2_ref[...][None, None, :], (bn, NS, C))
    for d in range(3):
        prh_d = jax.nn.relu(praw_refs[d][...] * ab1_ref[0:1, d:d + 1]
                            + ab1_ref[1:2, d:d + 1])
        pr = pr + prh_d[:, :, None] * wp2t_ref[d:d + 1, :][None, :, :]
    return pr.reshape(bn * NS, C)


def _wstats_kernel(p0_ref, p1_ref, p2_ref, gk_ref, xq_ref, wp2t_ref, bp2_ref,
                   ab1_ref, acc_ref):
    bn = gk_ref.shape[0]
    pr = _pr_block((p0_ref, p1_ref, p2_ref), ab1_ref, wp2t_ref, bp2_ref, bn)
    gk = gk_ref[...].reshape(bn * NS, C)
    xq = xq_ref[...]
    w_raw = gk - jnp.broadcast_to(xq[:, None, :], (bn, NS, C)).reshape(bn * NS, C) + pr

    @pl.when(pl.program_id(0) == 0)
    def _():
        acc_ref[...] = jnp.zeros_like(acc_ref)

    acc_ref[...] += jnp.stack([w_raw.sum(axis=0), (w_raw * w_raw).sum(axis=0)])


def _w1_kernel(p0_ref, p1_ref, p2_ref, gk_ref, xq_ref, wp2t_ref, bp2_ref,
               ab1_ref, ab2_ref, ww1_ref, bw1_ref, w1_ref, acc_ref):
    bn = gk_ref.shape[0]
    pr = _pr_block((p0_ref, p1_ref, p2_ref), ab1_ref, wp2t_ref, bp2_ref, bn)
    gk = gk_ref[...].reshape(bn * NS, C)
    xq = xq_ref[...]
    w_raw = gk - jnp.broadcast_to(xq[:, None, :], (bn, NS, C)).reshape(bn * NS, C) + pr
    wh = jax.nn.relu(w_raw * ab2_ref[0:1, :] + ab2_ref[1:2, :])
    w1 = jnp.dot(wh, ww1_ref[...].T, precision=HIGH) + bw1_ref[...]
    w1_ref[...] = w1.reshape(bn, NS, CS)

    @pl.when(pl.program_id(0) == 0)
    def _():
        acc_ref[...] = jnp.zeros_like(acc_ref)

    acc_ref[...] += jnp.stack([w1.sum(axis=0), (w1 * w1).sum(axis=0)])


def _outsum_kernel(w1_ref, gv_ref, p0_ref, p1_ref, p2_ref, wp2t_ref, bp2_ref,
                   ab1_ref, ab3_ref, ww2_ref, bw2_ref, o_ref):
    bn = w1_ref.shape[0]
    w1 = w1_ref[...].reshape(bn * NS, CS)
    wh = jax.nn.relu(w1 * ab3_ref[0:1, :] + ab3_ref[1:2, :])
    w2 = jnp.dot(wh, ww2_ref[...].T, precision=HIGH) + bw2_ref[...]
    w2 = w2.reshape(bn, NS, CS)
    w2 = w2 - jnp.max(w2, axis=1, keepdims=True)
    e = jnp.exp(w2)
    w = e / jnp.sum(e, axis=1, keepdims=True)

    pr = _pr_block((p0_ref, p1_ref, p2_ref), ab1_ref, wp2t_ref, bp2_ref, bn)
    t = (gv_ref[...].reshape(bn * NS, C) + pr).reshape(bn, NS, S, CS) * w[:, :, None, :]
    o_ref[...] = t.sum(axis=1).reshape(bn, C)


def _affine(acc, gamma, beta, eps=1e-5):
    mean = acc[0] / M_STAT
    var = acc[1] / M_STAT - mean * mean
    a = gamma / jnp.sqrt(var + eps)
    return jnp.stack([a, beta - mean * a])


def kernel(p, x, o, edges, boundary, Wq, bq, Wk, bk, Wv, bv, Wp1, bp1, gp1,
           betap1, Wp2, bp2, gw1, betaw1, Ww1, bw1l, gw2, betaw2, Ww2, bw2l):
    f32 = jnp.float32

    BP = 1024
    xq, xk, xv = pl.pallas_call(
        _proj_kernel,
        grid=(N // BP,),
        in_specs=[
            pl.BlockSpec((BP, C), lambda i: (i, 0)),
            pl.BlockSpec((C, C), lambda i: (0, 0)),
            pl.BlockSpec((C,), lambda i: (0,)),
            pl.BlockSpec((C, C), lambda i: (0, 0)),
            pl.BlockSpec((C,), lambda i: (0,)),
            pl.BlockSpec((C, C), lambda i: (0, 0)),
            pl.BlockSpec((C,), lambda i: (0,)),
        ],
        out_specs=[pl.BlockSpec((BP, C), lambda i: (i, 0))] * 3,
        out_shape=[jax.ShapeDtypeStruct((N, C), f32)] * 3,
    )(x, Wq, bq, Wk, bk, Wv, bv)

    pp = jnp.pad(p, ((0, 0), (0, 5)))
    sq = (p * p).sum(-1)
    BR = 256
    d2, gm = pl.pallas_call(
        _knnprep_kernel,
        grid=(N // BR,),
        in_specs=[
            pl.BlockSpec((BR, 8), lambda i: (i, 0)),
            pl.BlockSpec((8, N), lambda i: (0, 0)),
            pl.BlockSpec((BR,), lambda i: (i,)),
            pl.BlockSpec((1, N), lambda i: (0, 0)),
        ],
        out_specs=[
            pl.BlockSpec((BR, N), lambda i: (i, 0)),
            pl.BlockSpec((BR, NG), lambda i: (i, 0)),
        ],
        out_shape=[
            jax.ShapeDtypeStruct((N, N), f32),
            jax.ShapeDtypeStruct((N, NG), f32),
        ],
    )(pp, pp.T, sq, sq[None, :])
    idx = _sc_select(gm, d2)

    gp = p[idx]
    gx = gp[:, :, 0] - p[:, 0:1]
    gy = gp[:, :, 1] - p[:, 1:2]
    gz = gp[:, :, 2] - p[:, 2:3]
    gk = xk[idx]
    gv = xv[idx]
    Wp1f = Wp1.astype(f32)
    bp1c = jnp.broadcast_to(bp1[:, None], (3, 3)).astype(f32)
    Wp2T = Wp2.T

    BA = 2048
    nsspec = lambda: pl.BlockSpec((BA, NS), lambda i: (i, 0))
    p0, p1, p2, accA = pl.pallas_call(
        _prstats_kernel,
        grid=(N // BA,),
        in_specs=[
            nsspec(), nsspec(), nsspec(),
            pl.BlockSpec((3, 3), lambda i: (0, 0)),
            pl.BlockSpec((3, 3), lambda i: (0, 0)),
        ],
        out_specs=[
            nsspec(), nsspec(), nsspec(),
            pl.BlockSpec((6, NS), lambda i: (0, 0)),
        ],
        out_shape=[jax.ShapeDtypeStruct((N, NS), f32)] * 3
        + [jax.ShapeDtypeStruct((6, NS), f32)],
    )(gx, gy, gz, Wp1f, bp1c)
    acc1 = jnp.stack([accA[0::2].sum(axis=1), accA[1::2].sum(axis=1)], axis=0)
    ab1 = _affine(acc1, gp1, betap1)

    BB = 256
    pspec = lambda: pl.BlockSpec((BB, NS), lambda i: (i, 0))
    acc2 = pl.pallas_call(
        _wstats_kernel,
        grid=(N // BB,),
        in_specs=[
            pspec(), pspec(), pspec(),
            pl.BlockSpec((BB, NS, C), lambda i: (i, 0, 0)),
            pl.BlockSpec((BB, C), lambda i: (i, 0)),
            pl.BlockSpec((3, C), lambda i: (0, 0)),
            pl.BlockSpec((C,), lambda i: (0,)),
            pl.BlockSpec((2, 3), lambda i: (0, 0)),
        ],
        out_specs=pl.BlockSpec((2, C), lambda i: (0, 0)),
        out_shape=jax.ShapeDtypeStruct((2, C), f32),
    )(p0, p1, p2, gk, xq, Wp2T, bp2, ab1)
    ab2 = _affine(acc2, gw1, betaw1)

    w1, acc3 = pl.pallas_call(
        _w1_kernel,
        grid=(N // BB,),
        in_specs=[
            pspec(), pspec(), pspec(),
            pl.BlockSpec((BB, NS, C), lambda i: (i, 0, 0)),
            pl.BlockSpec((BB, C), lambda i: (i, 0)),
            pl.BlockSpec((3, C), lambda i: (0, 0)),
            pl.BlockSpec((C,), lambda i: (0,)),
            pl.BlockSpec((2, 3), lambda i: (0, 0)),
            pl.BlockSpec((2, C), lambda i: (0, 0)),
            pl.BlockSpec((CS, C), lambda i: (0, 0)),
            pl.BlockSpec((CS,), lambda i: (0,)),
        ],
        out_specs=[
            pl.BlockSpec((BB, NS, CS), lambda i: (i, 0, 0)),
            pl.BlockSpec((2, CS), lambda i: (0, 0)),
        ],
        out_shape=[
            jax.ShapeDtypeStruct((N, NS, CS), f32),
            jax.ShapeDtypeStruct((2, CS), f32),
        ],
    )(p0, p1, p2, gk, xq, Wp2T, bp2, ab1, ab2, Ww1, bw1l)
    ab3 = _affine(acc3, gw2, betaw2)

    out = pl.pallas_call(
        _outsum_kernel,
        grid=(N // BB,),
        in_specs=[
            pl.BlockSpec((BB, NS, CS), lambda i: (i, 0, 0)),
            pl.BlockSpec((BB, NS, C), lambda i: (i, 0, 0)),
            pspec(), pspec(), pspec(),
            pl.BlockSpec((3, C), lambda i: (0, 0)),
            pl.BlockSpec((C,), lambda i: (0,)),
            pl.BlockSpec((2, 3), lambda i: (0, 0)),
            pl.BlockSpec((2, CS), lambda i: (0, 0)),
            pl.BlockSpec((CS, CS), lambda i: (0, 0)),
            pl.BlockSpec((CS,), lambda i: (0,)),
        ],
        out_specs=pl.BlockSpec((BB, C), lambda i: (i, 0)),
        out_shape=jax.ShapeDtypeStruct((N, C), f32),
    )(w1, gv, p0, p1, p2, Wp2T, bp2, ab1, ab3, Ww2, bw2l)
    return out


# speedup vs baseline: 3.2820x; 1.1612x over previous
import functools

import jax
import jax.numpy as jnp
from jax import lax
from jax.experimental import pallas as pl
from jax.experimental.pallas import tpu as pltpu
from jax.experimental.pallas import tpu_sc as plsc

N = 8192
C = 256
NS = 16
S = 8
CS = C // S
M_STAT = N * NS

HIGH = jax.lax.Precision.HIGHEST


NG = 128
G = 64


def _knnprep_kernel(pp_ref, ppt_ref, sqc_ref, sqr_ref, d2_ref, gm_ref):
    br = pp_ref.shape[0]
    mm = jnp.dot(pp_ref[...].astype(jnp.bfloat16),
                 ppt_ref[...].astype(jnp.bfloat16),
                 preferred_element_type=jnp.float32)
    d2 = (sqc_ref[...][:, None] + sqr_ref[...]) - 2.0 * mm
    d2_ref[...] = d2
    gm_ref[...] = jnp.min(d2.reshape(br, NG, G), axis=2)


_NC = 2
_NSUB = 16
_NW = _NC * _NSUB
_RW = N // _NW


def _merge16(av, ai, bv, bi, resort=True):
    rbv = lax.rev(bv, (0,))
    rbi = lax.rev(bi, (0,))
    m = av <= rbv
    lov = jnp.where(m, av, rbv)
    loi = jnp.where(m, ai, rbi)
    if resort:
        sv, si = plsc.sort_key_val(lov, loi)
        return sv, si
    return lov, loi


def _sc_select_kernel(gm_hbm, d2s_hbm, xk_hbm, xv_hbm, pp16_hbm,
                      idx_hbm, gk_hbm, gv_hbm, gp_hbm,
                      gm_v, slab_v, grp_v, gidx_v, idx_v, cand_v,
                      gk_b, gv_b, gp_b,
                      sem, sgk, sgv, sgp, sok, sov, sop):
    nc = lax.axis_index("c")
    ns = lax.axis_index("s")
    wid = ns * _NC + nc
    base = wid * _RW
    iota = lax.broadcasted_iota(jnp.int32, (16,), 0)

    pltpu.sync_copy(gm_hbm.at[pl.ds(base * NG, _RW * NG)], gm_v)

    def row_body(r, _):
        pairs = []
        for c in range(0, 8, 2):
            av, ai = plsc.sort_key_val(gm_v[pl.ds(r * NG + c * 16, 16)],
                                       c * 16 + iota)
            bv, bi = plsc.sort_key_val(gm_v[pl.ds(r * NG + (c + 1) * 16, 16)],
                                       (c + 1) * 16 + iota)
            pairs.append(_merge16(av, ai, bv, bi))
        m01 = _merge16(*pairs[0], *pairs[1])
        m23 = _merge16(*pairs[2], *pairs[3])
        gv_, gi = _merge16(*m01, *m23)
        grp_v[...] = gi
        gidx_v[...] = (base + r) * NG + gi
        pltpu.async_copy(d2s_hbm.at[gidx_v], slab_v, sem).wait()
        cv, ci = plsc.sort_key_val(slab_v[0, pl.ds(0, 16)], iota)
        for t in range(16):
            for c in range(4):
                if t == 0 and c == 0:
                    continue
                pos = t * 64 + c * 16
                ch = slab_v[t, pl.ds(c * 16, 16)]
                mx = jnp.max(cv)

                def _do(cv=cv, ci=ci, ch=ch, pos=pos):
                    sv, si = plsc.sort_key_val(ch, pos + iota)
                    mv, mi = _merge16(cv, ci, sv, si)
                    return mv, mi

                def _skip(cv=cv, ci=ci):
                    return cv, ci

                cv, ci = lax.cond(jnp.min(ch) < mx, _do, _skip)
        t_of = ci >> 6
        l_of = ci & 63
        gsel = plsc.load_gather(grp_v, [t_of])
        cand = gsel * G + l_of
        idx_v[pl.ds(r * NS, NS)] = cand
        cand_v[...] = cand
        @pl.when(r > 0)
        def _():
            pltpu.make_async_copy(gk_hbm.at[pl.ds(0, NS)], gk_b, sok).wait()
            pltpu.make_async_copy(gv_hbm.at[pl.ds(0, NS)], gv_b, sov).wait()
            pltpu.make_async_copy(gp_hbm.at[pl.ds(0, NS)], gp_b, sop).wait()
        pltpu.async_copy(xk_hbm.at[cand_v], gk_b, sgk).wait()
        pltpu.async_copy(xv_hbm.at[cand_v], gv_b, sgv).wait()
        pltpu.async_copy(pp16_hbm.at[cand_v], gp_b, sgp).wait()
        orow = (base + r) * NS
        pltpu.async_copy(gk_b, gk_hbm.at[pl.ds(orow, NS)], sok)
        pltpu.async_copy(gv_b, gv_hbm.at[pl.ds(orow, NS)], sov)
        pltpu.async_copy(gp_b, gp_hbm.at[pl.ds(orow, NS)], sop)
        return ()

    lax.fori_loop(0, _RW, row_body, ())
    pltpu.make_async_copy(gk_hbm.at[pl.ds(0, NS)], gk_b, sok).wait()
    pltpu.make_async_copy(gv_hbm.at[pl.ds(0, NS)], gv_b, sov).wait()
    pltpu.make_async_copy(gp_hbm.at[pl.ds(0, NS)], gp_b, sop).wait()
    pltpu.sync_copy(idx_v, idx_hbm.at[pl.ds(base * NS, _RW * NS)])


def _sc_select(gm, d2, xk, xv, pp16):
    d2s = d2.reshape(N * NG, G)
    gmf = gm.reshape(N * NG)
    mesh = plsc.VectorSubcoreMesh(core_axis_name="c", subcore_axis_name="s")
    f32 = jnp.float32
    fn = pl.kernel(
        _sc_select_kernel,
        out_type=[
            jax.ShapeDtypeStruct((N * NS,), jnp.int32),
            jax.ShapeDtypeStruct((N * NS, C), f32),
            jax.ShapeDtypeStruct((N * NS, C), f32),
            jax.ShapeDtypeStruct((N * NS, 16), f32),
        ],
        mesh=mesh,
        scratch_types=[
            pltpu.VMEM((_RW * NG,), f32),
            pltpu.VMEM((16, G), f32),
            pltpu.VMEM((16,), jnp.int32),
            pltpu.VMEM((16,), jnp.int32),
            pltpu.VMEM((_RW * NS,), jnp.int32),
            pltpu.VMEM((16,), jnp.int32),
            pltpu.VMEM((NS, C), f32),
            pltpu.VMEM((NS, C), f32),
            pltpu.VMEM((NS, 16), f32),
        ] + [pltpu.SemaphoreType.DMA] * 7,
        compiler_params=pltpu.CompilerParams(needs_layout_passes=False,
                                             use_tc_tiling_on_sc=False),
    )
    idx, gk, gv, gp = fn(gmf, d2s, xk, xv, pp16)
    return (idx.reshape(N, NS), gk.reshape(N, NS, C), gv.reshape(N, NS, C),
            gp.reshape(N, NS, 16))


def _proj_kernel(x_ref, wq_ref, bq_ref, wk_ref, bk_ref, wv_ref, bv_ref,
                 xq_ref, xk_ref, xv_ref):
    x = x_ref[...]
    xq_ref[...] = jnp.dot(x, wq_ref[...].T, precision=HIGH) + bq_ref[...]
    xk_ref[...] = jnp.dot(x, wk_ref[...].T, precision=HIGH) + bk_ref[...]
    xv_ref[...] = jnp.dot(x, wv_ref[...].T, precision=HIGH) + bv_ref[...]


def _prstats_kernel(gx_ref, gy_ref, gz_ref, wp1_ref, bp1_ref,
                    p0_ref, p1_ref, p2_ref, acc_ref):
    g = (gx_ref[...], gy_ref[...], gz_ref[...])
    outs = (p0_ref, p1_ref, p2_ref)

    @pl.when(pl.program_id(0) == 0)
    def _():
        acc_ref[...] = jnp.zeros_like(acc_ref)

    for c in range(3):
        praw = bp1_ref[c:c + 1, 0:1]
        for d in range(3):
            praw = praw + g[d] * wp1_ref[c:c + 1, d:d + 1]
        outs[c][...] = praw
        acc_ref[2 * c:2 * c + 1, :] += praw.sum(axis=0, keepdims=True)
        acc_ref[2 * c + 1:2 * c + 2, :] += (praw * praw).sum(axis=0, keepdims=True)


def _pr_block(praw_refs, ab1_ref, wp2t_ref, bp2_ref, bn):
    pr = jnp.broadcast_to(bp2_ref[...][None, None, :], (bn, NS, C))
    for d in range(3):
        prh_d = jax.nn.relu(praw_refs[d][...] * ab1_ref[0:1, d:d + 1]
                            + ab1_ref[1:2, d:d + 1])
        pr = pr + prh_d[:, :, None] * wp2t_ref[d:d + 1, :][None, :, :]
    return pr.reshape(bn * NS, C)


def _wstats_kernel(p0_ref, p1_ref, p2_ref, gk_ref, xq_ref, wp2t_ref, bp2_ref,
                   ab1_ref, acc_ref):
    bn = gk_ref.shape[0]
    pr = _pr_block((p0_ref, p1_ref, p2_ref), ab1_ref, wp2t_ref, bp2_ref, bn)
    gk = gk_ref[...].reshape(bn * NS, C)
    xq = xq_ref[...]
    w_raw = gk - jnp.broadcast_to(xq[:, None, :], (bn, NS, C)).reshape(bn * NS, C) + pr

    @pl.when(pl.program_id(0) == 0)
    def _():
        acc_ref[...] = jnp.zeros_like(acc_ref)

    acc_ref[...] += jnp.stack([w_raw.sum(axis=0), (w_raw * w_raw).sum(axis=0)])


def _w1_kernel(p0_ref, p1_ref, p2_ref, gk_ref, xq_ref, wp2t_ref, bp2_ref,
               ab1_ref, ab2_ref, ww1_ref, bw1_ref, w1_ref, acc_ref):
    bn = gk_ref.shape[0]
    pr = _pr_block((p0_ref, p1_ref, p2_ref), ab1_ref, wp2t_ref, bp2_ref, bn)
    gk = gk_ref[...].reshape(bn * NS, C)
    xq = xq_ref[...]
    w_raw = gk - jnp.broadcast_to(xq[:, None, :], (bn, NS, C)).reshape(bn * NS, C) + pr
    wh = jax.nn.relu(w_raw * ab2_ref[0:1, :] + ab2_ref[1:2, :])
    w1 = jnp.dot(wh, ww1_ref[...].T, precision=HIGH) + bw1_ref[...]
    w1_ref[...] = w1.reshape(bn, NS, CS)

    @pl.when(pl.program_id(0) == 0)
    def _():
        acc_ref[...] = jnp.zeros_like(acc_ref)

    acc_ref[...] += jnp.stack([w1.sum(axis=0), (w1 * w1).sum(axis=0)])


def _outsum_kernel(w1_ref, gv_ref, p0_ref, p1_ref, p2_ref, wp2t_ref, bp2_ref,
                   ab1_ref, ab3_ref, ww2_ref, bw2_ref, o_ref):
    bn = w1_ref.shape[0]
    w1 = w1_ref[...].reshape(bn * NS, CS)
    wh = jax.nn.relu(w1 * ab3_ref[0:1, :] + ab3_ref[1:2, :])
    w2 = jnp.dot(wh, ww2_ref[...].T, precision=HIGH) + bw2_ref[...]
    w2 = w2.reshape(bn, NS, CS)
    w2 = w2 - jnp.max(w2, axis=1, keepdims=True)
    e = jnp.exp(w2)
    w = e / jnp.sum(e, axis=1, keepdims=True)

    pr = _pr_block((p0_ref, p1_ref, p2_ref), ab1_ref, wp2t_ref, bp2_ref, bn)
    t = (gv_ref[...].reshape(bn * NS, C) + pr).reshape(bn, NS, S, CS) * w[:, :, None, :]
    o_ref[...] = t.sum(axis=1).reshape(bn, C)


def _affine(acc, gamma, beta, eps=1e-5):
    mean = acc[0] / M_STAT
    var = acc[1] / M_STAT - mean * mean
    a = gamma / jnp.sqrt(var + eps)
    return jnp.stack([a, beta - mean * a])


def kernel(p, x, o, edges, boundary, Wq, bq, Wk, bk, Wv, bv, Wp1, bp1, gp1,
           betap1, Wp2, bp2, gw1, betaw1, Ww1, bw1l, gw2, betaw2, Ww2, bw2l):
    f32 = jnp.float32

    BP = 1024
    xq, xk, xv = pl.pallas_call(
        _proj_kernel,
        grid=(N // BP,),
        in_specs=[
            pl.BlockSpec((BP, C), lambda i: (i, 0)),
            pl.BlockSpec((C, C), lambda i: (0, 0)),
            pl.BlockSpec((C,), lambda i: (0,)),
            pl.BlockSpec((C, C), lambda i: (0, 0)),
            pl.BlockSpec((C,), lambda i: (0,)),
            pl.BlockSpec((C, C), lambda i: (0, 0)),
            pl.BlockSpec((C,), lambda i: (0,)),
        ],
        out_specs=[pl.BlockSpec((BP, C), lambda i: (i, 0))] * 3,
        out_shape=[jax.ShapeDtypeStruct((N, C), f32)] * 3,
    )(x, Wq, bq, Wk, bk, Wv, bv)

    pp = jnp.pad(p, ((0, 0), (0, 5)))
    sq = (p * p).sum(-1)
    BR = 256
    d2, gm = pl.pallas_call(
        _knnprep_kernel,
        grid=(N // BR,),
        in_specs=[
            pl.BlockSpec((BR, 8), lambda i: (i, 0)),
            pl.BlockSpec((8, N), lambda i: (0, 0)),
            pl.BlockSpec((BR,), lambda i: (i,)),
            pl.BlockSpec((1, N), lambda i: (0, 0)),
        ],
        out_specs=[
            pl.BlockSpec((BR, N), lambda i: (i, 0)),
            pl.BlockSpec((BR, NG), lambda i: (i, 0)),
        ],
        out_shape=[
            jax.ShapeDtypeStruct((N, N), f32),
            jax.ShapeDtypeStruct((N, NG), f32),
        ],
    )(pp, pp.T, sq, sq[None, :])
    pp16 = jnp.pad(p, ((0, 0), (0, 13)))
    idx, gk, gv, gp = _sc_select(gm, d2, xk, xv, pp16)

    gx = gp[:, :, 0] - p[:, 0:1]
    gy = gp[:, :, 1] - p[:, 1:2]
    gz = gp[:, :, 2] - p[:, 2:3]
    Wp1f = Wp1.astype(f32)
    bp1c = jnp.broadcast_to(bp1[:, None], (3, 3)).astype(f32)
    Wp2T = Wp2.T

    BA = 2048
    nsspec = lambda: pl.BlockSpec((BA, NS), lambda i: (i, 0))
    p0, p1, p2, accA = pl.pallas_call(
        _prstats_kernel,
        grid=(N // BA,),
        in_specs=[
            nsspec(), nsspec(), nsspec(),
            pl.BlockSpec((3, 3), lambda i: (0, 0)),
            pl.BlockSpec((3, 3), lambda i: (0, 0)),
        ],
        out_specs=[
            nsspec(), nsspec(), nsspec(),
            pl.BlockSpec((6, NS), lambda i: (0, 0)),
        ],
        out_shape=[jax.ShapeDtypeStruct((N, NS), f32)] * 3
        + [jax.ShapeDtypeStruct((6, NS), f32)],
    )(gx, gy, gz, Wp1f, bp1c)
    acc1 = jnp.stack([accA[0::2].sum(axis=1), accA[1::2].sum(axis=1)], axis=0)
    ab1 = _affine(acc1, gp1, betap1)

    BB = 256
    pspec = lambda: pl.BlockSpec((BB, NS), lambda i: (i, 0))
    acc2 = pl.pallas_call(
        _wstats_kernel,
        grid=(N // BB,),
        in_specs=[
            pspec(), pspec(), pspec(),
            pl.BlockSpec((BB, NS, C), lambda i: (i, 0, 0)),
            pl.BlockSpec((BB, C), lambda i: (i, 0)),
            pl.BlockSpec((3, C), lambda i: (0, 0)),
            pl.BlockSpec((C,), lambda i: (0,)),
            pl.BlockSpec((2, 3), lambda i: (0, 0)),
        ],
        out_specs=pl.BlockSpec((2, C), lambda i: (0, 0)),
        out_shape=jax.ShapeDtypeStruct((2, C), f32),
    )(p0, p1, p2, gk, xq, Wp2T, bp2, ab1)
    ab2 = _affine(acc2, gw1, betaw1)

    w1, acc3 = pl.pallas_call(
        _w1_kernel,
        grid=(N // BB,),
        in_specs=[
            pspec(), pspec(), pspec(),
            pl.BlockSpec((BB, NS, C), lambda i: (i, 0, 0)),
            pl.BlockSpec((BB, C), lambda i: (i, 0)),
            pl.BlockSpec((3, C), lambda i: (0, 0)),
            pl.BlockSpec((C,), lambda i: (0,)),
            pl.BlockSpec((2, 3), lambda i: (0, 0)),
            pl.BlockSpec((2, C), lambda i: (0, 0)),
            pl.BlockSpec((CS, C), lambda i: (0, 0)),
            pl.BlockSpec((CS,), lambda i: (0,)),
        ],
        out_specs=[
            pl.BlockSpec((BB, NS, CS), lambda i: (i, 0, 0)),
            pl.BlockSpec((2, CS), lambda i: (0, 0)),
        ],
        out_shape=[
            jax.ShapeDtypeStruct((N, NS, CS), f32),
            jax.ShapeDtypeStruct((2, CS), f32),
        ],
    )(p0, p1, p2, gk, xq, Wp2T, bp2, ab1, ab2, Ww1, bw1l)
    ab3 = _affine(acc3, gw2, betaw2)

    out = pl.pallas_call(
        _outsum_kernel,
        grid=(N // BB,),
        in_specs=[
            pl.BlockSpec((BB, NS, CS), lambda i: (i, 0, 0)),
            pl.BlockSpec((BB, NS, C), lambda i: (i, 0, 0)),
            pspec(), pspec(), pspec(),
            pl.BlockSpec((3, C), lambda i: (0, 0)),
            pl.BlockSpec((C,), lambda i: (0,)),
            pl.BlockSpec((2, 3), lambda i: (0, 0)),
            pl.BlockSpec((2, CS), lambda i: (0, 0)),
            pl.BlockSpec((CS, CS), lambda i: (0, 0)),
            pl.BlockSpec((CS,), lambda i: (0,)),
        ],
        out_specs=pl.BlockSpec((BB, C), lambda i: (i, 0)),
        out_shape=jax.ShapeDtypeStruct((N, C), f32),
    )(w1, gv, p0, p1, p2, Wp2T, bp2, ab1, ab3, Ww2, bw2l)
    return out


# speedup vs baseline: 3.7554x; 1.1443x over previous
import functools

import jax
import jax.numpy as jnp
from jax import lax
from jax.experimental import pallas as pl
from jax.experimental.pallas import tpu as pltpu
from jax.experimental.pallas import tpu_sc as plsc

N = 8192
C = 256
NS = 16
S = 8
CS = C // S
M_STAT = N * NS

HIGH = jax.lax.Precision.HIGHEST


NG = 128
G = 64


def _knnprep_kernel(pp_ref, ppt_ref, sqc_ref, sqr_ref, d2_ref, gm_ref):
    br = pp_ref.shape[0]
    mm = jnp.dot(pp_ref[...].astype(jnp.bfloat16),
                 ppt_ref[...].astype(jnp.bfloat16),
                 preferred_element_type=jnp.float32)
    d2 = (sqc_ref[...][:, None] + sqr_ref[...]) - 2.0 * mm
    d2_ref[...] = d2
    gm_ref[...] = jnp.min(d2.reshape(br, NG, G), axis=2)


_NC = 2
_NSUB = 16
_NW = _NC * _NSUB
_RW = N // _NW


def _merge16(av, ai, bv, bi, resort=True):
    rbv = lax.rev(bv, (0,))
    rbi = lax.rev(bi, (0,))
    m = (av < rbv) | ((av == rbv) & (ai <= rbi))
    lov = jnp.where(m, av, rbv)
    loi = jnp.where(m, ai, rbi)
    if resort:
        sv, si = plsc.sort_key_val(lov, loi)
        return sv, si
    return lov, loi


def _sc_select_kernel(gm_hbm, d2s_hbm, xk_hbm, xv_hbm, pp16_hbm,
                      idx_hbm, gk_hbm, gv_hbm, gp_hbm,
                      gm_v, slab_v, grp_v, gidx_v, idx_v, cand_v,
                      gk_b, gv_b, gp_b,
                      ssl0, ssl1, sg, so0, so1):
    nc = lax.axis_index("c")
    ns = lax.axis_index("s")
    wid = ns * _NC + nc
    base = wid * _RW
    iota = lax.broadcasted_iota(jnp.int32, (16,), 0)
    ssl = (ssl0, ssl1)
    so = (so0, so1)

    pltpu.sync_copy(gm_hbm.at[pl.ds(base * NG, _RW * NG)], gm_v)

    def stage1(r, nb):
        pairs = []
        for c in range(0, 8, 2):
            av, ai = plsc.sort_key_val(gm_v[pl.ds(r * NG + c * 16, 16)],
                                       c * 16 + iota)
            bv, bi = plsc.sort_key_val(gm_v[pl.ds(r * NG + (c + 1) * 16, 16)],
                                       (c + 1) * 16 + iota)
            pairs.append(_merge16(av, ai, bv, bi))
        m01 = _merge16(*pairs[0], *pairs[1])
        m23 = _merge16(*pairs[2], *pairs[3])
        gv_, gi = _merge16(*m01, *m23)
        grp_v.at[nb][...] = gi
        gidx_v.at[nb][...] = (base + r) * NG + gi
        pltpu.async_copy(d2s_hbm.at[gidx_v.at[nb]], slab_v.at[nb], ssl[nb])

    def do_iter(r, b):
        nb = 1 - b

        @pl.when(r + 1 < _RW)
        def _():
            stage1(r + 1, nb)

        pltpu.make_async_copy(d2s_hbm.at[pl.ds(0, 16)], slab_v.at[b],
                              ssl[b]).wait()
        sl = slab_v.at[b]
        cv, ci = plsc.sort_key_val(sl[0, pl.ds(0, 16)], iota)
        cmax = jnp.max(cv)
        for t in range(16):
            for c in range(4):
                if t == 0 and c == 0:
                    continue
                pos = t * 64 + c * 16
                ch = sl[t, pl.ds(c * 16, 16)]

                def _do(cv=cv, ci=ci, ch=ch, pos=pos):
                    sv, si = plsc.sort_key_val(ch, pos + iota)
                    mv, mi = _merge16(cv, ci, sv, si)
                    return mv, mi, jnp.max(mv)

                def _skip(cv=cv, ci=ci, cmax=cmax):
                    return cv, ci, cmax

                cv, ci, cmax = lax.cond(jnp.any(ch < cmax), _do, _skip)
        t_of = ci >> 6
        l_of = ci & 63
        gsel = plsc.load_gather(grp_v.at[b], [t_of])
        cand = gsel * G + l_of
        idx_v[pl.ds(r * NS, NS)] = cand

        @pl.when(r >= 1)
        def _():
            pltpu.make_async_copy(xk_hbm.at[pl.ds(0, NS)], gk_b.at[nb], sg).wait()
            pltpu.make_async_copy(xv_hbm.at[pl.ds(0, NS)], gv_b.at[nb], sg).wait()
            pltpu.make_async_copy(pp16_hbm.at[pl.ds(0, NS)], gp_b.at[nb], sg).wait()
            orow = (base + r - 1) * NS
            pltpu.async_copy(gk_b.at[nb], gk_hbm.at[pl.ds(orow, NS)], so[nb])
            pltpu.async_copy(gv_b.at[nb], gv_hbm.at[pl.ds(orow, NS)], so[nb])
            pltpu.async_copy(gp_b.at[nb], gp_hbm.at[pl.ds(orow, NS)], so[nb])

        @pl.when(r >= 2)
        def _():
            pltpu.make_async_copy(xk_hbm.at[pl.ds(0, NS)], gk_b.at[b], so[b]).wait()
            pltpu.make_async_copy(xv_hbm.at[pl.ds(0, NS)], gv_b.at[b], so[b]).wait()
            pltpu.make_async_copy(pp16_hbm.at[pl.ds(0, NS)], gp_b.at[b], so[b]).wait()

        cand_v.at[b][...] = cand
        pltpu.async_copy(xk_hbm.at[cand_v.at[b]], gk_b.at[b], sg)
        pltpu.async_copy(xv_hbm.at[cand_v.at[b]], gv_b.at[b], sg)
        pltpu.async_copy(pp16_hbm.at[cand_v.at[b]], gp_b.at[b], sg)

    stage1(0, 0)

    def pair_body(i, _):
        do_iter(2 * i, 0)
        do_iter(2 * i + 1, 1)
        return ()

    lax.fori_loop(0, _RW // 2, pair_body, ())

    last = _RW - 1
    pltpu.make_async_copy(xk_hbm.at[pl.ds(0, NS)], gk_b.at[1], sg).wait()
    pltpu.make_async_copy(xv_hbm.at[pl.ds(0, NS)], gv_b.at[1], sg).wait()
    pltpu.make_async_copy(pp16_hbm.at[pl.ds(0, NS)], gp_b.at[1], sg).wait()
    orow = (base + last) * NS
    pltpu.async_copy(gk_b.at[1], gk_hbm.at[pl.ds(orow, NS)], so[1])
    pltpu.async_copy(gv_b.at[1], gv_hbm.at[pl.ds(orow, NS)], so[1])
    pltpu.async_copy(gp_b.at[1], gp_hbm.at[pl.ds(orow, NS)], so[1])
    for b in (0, 1):
        pltpu.make_async_copy(xk_hbm.at[pl.ds(0, NS)], gk_b.at[b], so[b]).wait()
        pltpu.make_async_copy(xv_hbm.at[pl.ds(0, NS)], gv_b.at[b], so[b]).wait()
        pltpu.make_async_copy(pp16_hbm.at[pl.ds(0, NS)], gp_b.at[b], so[b]).wait()
    pltpu.sync_copy(idx_v, idx_hbm.at[pl.ds(base * NS, _RW * NS)])


def _sc_select(gm, d2, xk, xv, pp16):
    d2s = d2.reshape(N * NG, G)
    gmf = gm.reshape(N * NG)
    mesh = plsc.VectorSubcoreMesh(core_axis_name="c", subcore_axis_name="s")
    f32 = jnp.float32
    fn = pl.kernel(
        _sc_select_kernel,
        out_type=[
            jax.ShapeDtypeStruct((N * NS,), jnp.int32),
            jax.ShapeDtypeStruct((N * NS, C), f32),
            jax.ShapeDtypeStruct((N * NS, C), f32),
            jax.ShapeDtypeStruct((N * NS, 16), f32),
        ],
        mesh=mesh,
        scratch_types=[
            pltpu.VMEM((_RW * NG,), f32),
            pltpu.VMEM((2, 16, G), f32),
            pltpu.VMEM((2, 16), jnp.int32),
            pltpu.VMEM((2, 16), jnp.int32),
            pltpu.VMEM((_RW * NS,), jnp.int32),
            pltpu.VMEM((2, 16), jnp.int32),
            pltpu.VMEM((2, NS, C), f32),
            pltpu.VMEM((2, NS, C), f32),
            pltpu.VMEM((2, NS, 16), f32),
        ] + [pltpu.SemaphoreType.DMA] * 5,
        compiler_params=pltpu.CompilerParams(needs_layout_passes=False,
                                             use_tc_tiling_on_sc=False),
    )
    idx, gk, gv, gp = fn(gmf, d2s, xk, xv, pp16)
    return (idx.reshape(N, NS), gk.reshape(N, NS, C), gv.reshape(N, NS, C),
            gp.reshape(N, NS, 16))


def _proj_kernel(x_ref, wq_ref, bq_ref, wk_ref, bk_ref, wv_ref, bv_ref,
                 xq_ref, xk_ref, xv_ref):
    x = x_ref[...]
    xq_ref[...] = jnp.dot(x, wq_ref[...].T, precision=HIGH) + bq_ref[...]
    xk_ref[...] = jnp.dot(x, wk_ref[...].T, precision=HIGH) + bk_ref[...]
    xv_ref[...] = jnp.dot(x, wv_ref[...].T, precision=HIGH) + bv_ref[...]


def _prstats_kernel(gx_ref, gy_ref, gz_ref, wp1_ref, bp1_ref,
                    p0_ref, p1_ref, p2_ref, acc_ref):
    g = (gx_ref[...], gy_ref[...], gz_ref[...])
    outs = (p0_ref, p1_ref, p2_ref)

    @pl.when(pl.program_id(0) == 0)
    def _():
        acc_ref[...] = jnp.zeros_like(acc_ref)

    for c in range(3):
        praw = bp1_ref[c:c + 1, 0:1]
        for d in range(3):
            praw = praw + g[d] * wp1_ref[c:c + 1, d:d + 1]
        outs[c][...] = praw
        acc_ref[2 * c:2 * c + 1, :] += praw.sum(axis=0, keepdims=True)
        acc_ref[2 * c + 1:2 * c + 2, :] += (praw * praw).sum(axis=0, keepdims=True)


def _pr_block(praw_refs, ab1_ref, wp2t_ref, bp2_ref, bn):
    pr = jnp.broadcast_to(bp2_ref[...][None, None, :], (bn, NS, C))
    for d in range(3):
        prh_d = jax.nn.relu(praw_refs[d][...] * ab1_ref[0:1, d:d + 1]
                            + ab1_ref[1:2, d:d + 1])
        pr = pr + prh_d[:, :, None] * wp2t_ref[d:d + 1, :][None, :, :]
    return pr.reshape(bn * NS, C)


def _wstats_kernel(p0_ref, p1_ref, p2_ref, gk_ref, xq_ref, wp2t_ref, bp2_ref,
                   ab1_ref, acc_ref):
    bn = gk_ref.shape[0]
    pr = _pr_block((p0_ref, p1_ref, p2_ref), ab1_ref, wp2t_ref, bp2_ref, bn)
    gk = gk_ref[...].reshape(bn * NS, C)
    xq = xq_ref[...]
    w_raw = gk - jnp.broadcast_to(xq[:, None, :], (bn, NS, C)).reshape(bn * NS, C) + pr

    @pl.when(pl.program_id(0) == 0)
    def _():
        acc_ref[...] = jnp.zeros_like(acc_ref)

    acc_ref[...] += jnp.stack([w_raw.sum(axis=0), (w_raw * w_raw).sum(axis=0)])


def _w1_kernel(p0_ref, p1_ref, p2_ref, gk_ref, xq_ref, wp2t_ref, bp2_ref,
               ab1_ref, ab2_ref, ww1_ref, bw1_ref, w1_ref, acc_ref):
    bn = gk_ref.shape[0]
    pr = _pr_block((p0_ref, p1_ref, p2_ref), ab1_ref, wp2t_ref, bp2_ref, bn)
    gk = gk_ref[...].reshape(bn * NS, C)
    xq = xq_ref[...]
    w_raw = gk - jnp.broadcast_to(xq[:, None, :], (bn, NS, C)).reshape(bn * NS, C) + pr
    wh = jax.nn.relu(w_raw * ab2_ref[0:1, :] + ab2_ref[1:2, :])
    w1 = jnp.dot(wh, ww1_ref[...].T, precision=HIGH) + bw1_ref[...]
    w1_ref[...] = w1.reshape(bn, NS, CS)

    @pl.when(pl.program_id(0) == 0)
    def _():
        acc_ref[...] = jnp.zeros_like(acc_ref)

    acc_ref[...] += jnp.stack([w1.sum(axis=0), (w1 * w1).sum(axis=0)])


def _outsum_kernel(w1_ref, gv_ref, p0_ref, p1_ref, p2_ref, wp2t_ref, bp2_ref,
                   ab1_ref, ab3_ref, ww2_ref, bw2_ref, o_ref):
    bn = w1_ref.shape[0]
    w1 = w1_ref[...].reshape(bn * NS, CS)
    wh = jax.nn.relu(w1 * ab3_ref[0:1, :] + ab3_ref[1:2, :])
    w2 = jnp.dot(wh, ww2_ref[...].T, precision=HIGH) + bw2_ref[...]
    w2 = w2.reshape(bn, NS, CS)
    w2 = w2 - jnp.max(w2, axis=1, keepdims=True)
    e = jnp.exp(w2)
    w = e / jnp.sum(e, axis=1, keepdims=True)

    pr = _pr_block((p0_ref, p1_ref, p2_ref), ab1_ref, wp2t_ref, bp2_ref, bn)
    t = (gv_ref[...].reshape(bn * NS, C) + pr).reshape(bn, NS, S, CS) * w[:, :, None, :]
    o_ref[...] = t.sum(axis=1).reshape(bn, C)


def _affine(acc, gamma, beta, eps=1e-5):
    mean = acc[0] / M_STAT
    var = acc[1] / M_STAT - mean * mean
    a = gamma / jnp.sqrt(var + eps)
    return jnp.stack([a, beta - mean * a])


def kernel(p, x, o, edges, boundary, Wq, bq, Wk, bk, Wv, bv, Wp1, bp1, gp1,
           betap1, Wp2, bp2, gw1, betaw1, Ww1, bw1l, gw2, betaw2, Ww2, bw2l):
    f32 = jnp.float32

    BP = 1024
    xq, xk, xv = pl.pallas_call(
        _proj_kernel,
        grid=(N // BP,),
        in_specs=[
            pl.BlockSpec((BP, C), lambda i: (i, 0)),
            pl.BlockSpec((C, C), lambda i: (0, 0)),
            pl.BlockSpec((C,), lambda i: (0,)),
            pl.BlockSpec((C, C), lambda i: (0, 0)),
            pl.BlockSpec((C,), lambda i: (0,)),
            pl.BlockSpec((C, C), lambda i: (0, 0)),
            pl.BlockSpec((C,), lambda i: (0,)),
        ],
        out_specs=[pl.BlockSpec((BP, C), lambda i: (i, 0))] * 3,
        out_shape=[jax.ShapeDtypeStruct((N, C), f32)] * 3,
    )(x, Wq, bq, Wk, bk, Wv, bv)

    pp = jnp.pad(p, ((0, 0), (0, 5)))
    sq = (p * p).sum(-1)
    BR = 256
    d2, gm = pl.pallas_call(
        _knnprep_kernel,
        grid=(N // BR,),
        in_specs=[
            pl.BlockSpec((BR, 8), lambda i: (i, 0)),
            pl.BlockSpec((8, N), lambda i: (0, 0)),
            pl.BlockSpec((BR,), lambda i: (i,)),
            pl.BlockSpec((1, N), lambda i: (0, 0)),
        ],
        out_specs=[
            pl.BlockSpec((BR, N), lambda i: (i, 0)),
            pl.BlockSpec((BR, NG), lambda i: (i, 0)),
        ],
        out_shape=[
            jax.ShapeDtypeStruct((N, N), f32),
            jax.ShapeDtypeStruct((N, NG), f32),
        ],
    )(pp, pp.T, sq, sq[None, :])
    pp16 = jnp.pad(p, ((0, 0), (0, 13)))
    idx, gk, gv, gp = _sc_select(gm, d2, xk, xv, pp16)

    gx = gp[:, :, 0] - p[:, 0:1]
    gy = gp[:, :, 1] - p[:, 1:2]
    gz = gp[:, :, 2] - p[:, 2:3]
    Wp1f = Wp1.astype(f32)
    bp1c = jnp.broadcast_to(bp1[:, None], (3, 3)).astype(f32)
    Wp2T = Wp2.T

    BA = 2048
    nsspec = lambda: pl.BlockSpec((BA, NS), lambda i: (i, 0))
    p0, p1, p2, accA = pl.pallas_call(
        _prstats_kernel,
        grid=(N // BA,),
        in_specs=[
            nsspec(), nsspec(), nsspec(),
            pl.BlockSpec((3, 3), lambda i: (0, 0)),
            pl.BlockSpec((3, 3), lambda i: (0, 0)),
        ],
        out_specs=[
            nsspec(), nsspec(), nsspec(),
            pl.BlockSpec((6, NS), lambda i: (0, 0)),
        ],
        out_shape=[jax.ShapeDtypeStruct((N, NS), f32)] * 3
        + [jax.ShapeDtypeStruct((6, NS), f32)],
    )(gx, gy, gz, Wp1f, bp1c)
    acc1 = jnp.stack([accA[0::2].sum(axis=1), accA[1::2].sum(axis=1)], axis=0)
    ab1 = _affine(acc1, gp1, betap1)

    BB = 256
    pspec = lambda: pl.BlockSpec((BB, NS), lambda i: (i, 0))
    acc2 = pl.pallas_call(
        _wstats_kernel,
        grid=(N // BB,),
        in_specs=[
            pspec(), pspec(), pspec(),
            pl.BlockSpec((BB, NS, C), lambda i: (i, 0, 0)),
            pl.BlockSpec((BB, C), lambda i: (i, 0)),
            pl.BlockSpec((3, C), lambda i: (0, 0)),
            pl.BlockSpec((C,), lambda i: (0,)),
            pl.BlockSpec((2, 3), lambda i: (0, 0)),
        ],
        out_specs=pl.BlockSpec((2, C), lambda i: (0, 0)),
        out_shape=jax.ShapeDtypeStruct((2, C), f32),
    )(p0, p1, p2, gk, xq, Wp2T, bp2, ab1)
    ab2 = _affine(acc2, gw1, betaw1)

    w1, acc3 = pl.pallas_call(
        _w1_kernel,
        grid=(N // BB,),
        in_specs=[
            pspec(), pspec(), pspec(),
            pl.BlockSpec((BB, NS, C), lambda i: (i, 0, 0)),
            pl.BlockSpec((BB, C), lambda i: (i, 0)),
            pl.BlockSpec((3, C), lambda i: (0, 0)),
            pl.BlockSpec((C,), lambda i: (0,)),
            pl.BlockSpec((2, 3), lambda i: (0, 0)),
            pl.BlockSpec((2, C), lambda i: (0, 0)),
            pl.BlockSpec((CS, C), lambda i: (0, 0)),
            pl.BlockSpec((CS,), lambda i: (0,)),
        ],
        out_specs=[
            pl.BlockSpec((BB, NS, CS), lambda i: (i, 0, 0)),
            pl.BlockSpec((2, CS), lambda i: (0, 0)),
        ],
        out_shape=[
            jax.ShapeDtypeStruct((N, NS, CS), f32),
            jax.ShapeDtypeStruct((2, CS), f32),
        ],
    )(p0, p1, p2, gk, xq, Wp2T, bp2, ab1, ab2, Ww1, bw1l)
    ab3 = _affine(acc3, gw2, betaw2)

    out = pl.pallas_call(
        _outsum_kernel,
        grid=(N // BB,),
        in_specs=[
            pl.BlockSpec((BB, NS, CS), lambda i: (i, 0, 0)),
            pl.BlockSpec((BB, NS, C), lambda i: (i, 0, 0)),
            pspec(), pspec(), pspec(),
            pl.BlockSpec((3, C), lambda i: (0, 0)),
            pl.BlockSpec((C,), lambda i: (0,)),
            pl.BlockSpec((2, 3), lambda i: (0, 0)),
            pl.BlockSpec((2, CS), lambda i: (0, 0)),
            pl.BlockSpec((CS, CS), lambda i: (0, 0)),
            pl.BlockSpec((CS,), lambda i: (0,)),
        ],
        out_specs=pl.BlockSpec((BB, C), lambda i: (i, 0)),
        out_shape=jax.ShapeDtypeStruct((N, C), f32),
    )(w1, gv, p0, p1, p2, Wp2T, bp2, ab1, ab3, Ww2, bw2l)
    return out
